# Initial kernel scaffold; baseline (speedup 1.0000x reference)
import jax
import jax.numpy as jnp
from jax.experimental import pallas as pl


def kernel(node_features, edge_features, Esrc, Etgt, batch, Wgc_in, bgc_in, Wgc_mid, bgc_mid, Wgc_out, bgc_out, We1_in, be1_in, We2_in, be2_in, We1_mid, be1_mid, We2_mid, be2_mid, We1_out, be1_out, We2_out, be2_out):
    raise NotImplementedError("write your pallas kernel here")


# speedup vs baseline: 4.0830x; 4.0830x over previous
import functools

import jax
import jax.numpy as jnp
from jax import lax
from jax.experimental import pallas as pl
from jax.experimental.pallas import tpu as pltpu
from jax.experimental.pallas import tpu_sc as plsc

_N = 10000
_E = 320000
_DF = 128
_DE = 16
_H = 73
_HP = 128
_NG = 64

_NC = 2
_NS = 16
_NW = _NC * _NS
_EPW = _E // _NW
_K = 80
_NCHUNK = _EPW // _K


def _emlp_body(ef_ref, w1i, b1i, w2i, b2i, w1m, b1m, w2m, b2m,
               w1o, b1o, w2o, b2o, efin_ref, efmid_ref, efout_ref):
  ef = ef_ref[...]

  def mlp(w1, b1, w2, b2):
    h = jnp.maximum(
        jnp.dot(ef, w1[...], preferred_element_type=jnp.float32) + b1[...], 0.0)
    z = jnp.dot(h, w2[...], preferred_element_type=jnp.float32) + b2[...]
    return jax.nn.sigmoid(z)

  efin_ref[...] = mlp(w1i, b1i, w2i, b2i)
  efmid_ref[...] = mlp(w1m, b1m, w2m, b2m)
  efout_ref[...] = mlp(w1o, b1o, w2o, b2o)


def _edge_mlps(ef, w1i, b1i, w2i, b2i, w1m, b1m, w2m, b2m, w1o, b1o, w2o, b2o):
  be = 8000
  grid = _E // be
  full = lambda shape: pl.BlockSpec(shape, lambda i: (0, 0))
  return pl.pallas_call(
      _emlp_body,
      grid=(grid,),
      in_specs=[
          pl.BlockSpec((be, _DE), lambda i: (i, 0)),
          full(w1i.shape), full(b1i.shape), full(w2i.shape), full(b2i.shape),
          full(w1m.shape), full(b1m.shape), full(w2m.shape), full(b2m.shape),
          full(w1o.shape), full(b1o.shape), full(w2o.shape), full(b2o.shape),
      ],
      out_specs=[
          pl.BlockSpec((be, _HP), lambda i: (i, 0)),
          pl.BlockSpec((be, _HP), lambda i: (i, 0)),
          pl.BlockSpec((be, 1), lambda i: (i, 0)),
      ],
      out_shape=[
          jax.ShapeDtypeStruct((_E, _HP), jnp.float32),
          jax.ShapeDtypeStruct((_E, _HP), jnp.float32),
          jax.ShapeDtypeStruct((_E, 1), jnp.float32),
      ],
  )(ef, w1i, b1i, w2i, b2i, w1m, b1m, w2m, b2m, w1o, b1o, w2o, b2o)


def _lin_body(x_ref, w_ref, b_ref, o_ref):
  o_ref[...] = (jnp.dot(x_ref[...], w_ref[...],
                        preferred_element_type=jnp.float32) + b_ref[...])


def _linear(x, w, b):
  return pl.pallas_call(
      _lin_body,
      out_shape=jax.ShapeDtypeStruct((x.shape[0], w.shape[1]), jnp.float32),
  )(x, w, b)


def _mid_body(p_ref, w_ref, b_ref, o_ref):
  y = jnp.maximum(p_ref[0] + p_ref[1], 0.0)
  o_ref[...] = (jnp.dot(y, w_ref[...],
                        preferred_element_type=jnp.float32) + b_ref[...])


def _relu_sum_linear(p, w, b):
  return pl.pallas_call(
      _mid_body,
      out_shape=jax.ShapeDtypeStruct((p.shape[1], w.shape[1]), jnp.float32),
  )(p, w, b)


def _pool_body(b_ref, q_ref, o_ref):
  y = q_ref[0] + q_ref[1]
  g = lax.broadcasted_iota(jnp.int32, (_NG, _N), 0)
  m = (g == b_ref[...]).astype(jnp.float32)
  o_ref[...] = jnp.dot(m, y, preferred_element_type=jnp.float32)


def _pool(batch_row, q):
  return pl.pallas_call(
      _pool_body,
      out_shape=jax.ShapeDtypeStruct((_NG, 1), jnp.float32),
  )(batch_row, q)


_MESH = plsc.VectorSubcoreMesh(
    core_axis_name="c", subcore_axis_name="s", num_cores=_NC, num_subcores=_NS)


@functools.partial(
    pl.kernel,
    out_type=jax.ShapeDtypeStruct((_NC, _N, _HP), jnp.float32),
    mesh=_MESH,
    scratch_types=[
        pltpu.VMEM((_K,), jnp.int32),
        pltpu.VMEM((_K,), jnp.int32),
        pltpu.VMEM((_K, _HP), jnp.float32),
        pltpu.VMEM((_K, _HP), jnp.float32),
        pltpu.VMEM_SHARED((_N, _HP), jnp.float32),
    ],
)
def _sc_layer(sup_hbm, gate_hbm, esrc_hbm, etgt_hbm, zer_hbm, out_hbm,
              esrc_v, etgt_v, rows_v, gate_v, acc_sh):
  cid = lax.axis_index("c")
  sid = lax.axis_index("s")
  wid = cid * _NS + sid
  @pl.when(sid < 10)
  def _zero():
    pltpu.sync_copy(zer_hbm.at[pl.ds(sid * 1000, 1000)],
                    acc_sh.at[pl.ds(sid * 1000, 1000)])

  plsc.subcore_barrier()

  base = wid * _EPW

  def chunk(i, carry):
    e0 = base + i * _K
    pltpu.sync_copy(esrc_hbm.at[pl.ds(e0, _K)], esrc_v)
    pltpu.sync_copy(etgt_hbm.at[pl.ds(e0, _K)], etgt_v)
    pltpu.sync_copy(sup_hbm.at[esrc_v], rows_v)
    pltpu.sync_copy(gate_hbm.at[pl.ds(e0, _K)], gate_v)

    @plsc.parallel_loop(0, _K, unroll=4)
    def edge(j):
      for t in range(5):
        sl = pl.ds(t * 16, 16)
        rows_v[j, sl] = rows_v[j, sl] * gate_v[j, sl]
    pltpu.sync_copy(rows_v, acc_sh.at[etgt_v], add=True)
    return carry

  lax.fori_loop(0, _NCHUNK, chunk, 0)
  plsc.subcore_barrier()

  @pl.when(sid < 10)
  def _writeout():
    pltpu.sync_copy(acc_sh.at[pl.ds(sid * 1000, 1000)],
                    out_hbm.at[cid, pl.ds(sid * 1000, 1000)])


@functools.partial(
    pl.kernel,
    out_type=jax.ShapeDtypeStruct((_NC * _N,), jnp.float32),
    mesh=_MESH,
    compiler_params=pltpu.CompilerParams(needs_layout_passes=False),
    scratch_types=[
        pltpu.VMEM((_N,), jnp.float32),
        pltpu.VMEM((_K,), jnp.int32),
        pltpu.VMEM((_K,), jnp.int32),
        pltpu.VMEM((_K,), jnp.float32),
        pltpu.VMEM((_K,), jnp.float32),
        pltpu.VMEM((1000,), jnp.float32),
        pltpu.VMEM_SHARED((_N,), jnp.float32),
    ],
)
def _sc_layer_out(sup_hbm, gate_hbm, esrc_hbm, etgt_hbm, zer_hbm, out_hbm,
                  sup_v, esrc_v, etgt_v, gate_v, m_v, stage_v, acc_sh):
  cid = lax.axis_index("c")
  sid = lax.axis_index("s")
  wid = cid * _NS + sid
  pltpu.sync_copy(sup_hbm, sup_v)

  @pl.when(sid < 10)
  def _zero():
    pltpu.sync_copy(zer_hbm.at[pl.ds(sid * 1000, 1000)], stage_v)
    pltpu.sync_copy(stage_v, acc_sh.at[pl.ds(sid * 1000, 1000)])

  plsc.subcore_barrier()
  base = wid * _EPW

  def chunk(i, carry):
    e0 = base + i * _K
    pltpu.sync_copy(esrc_hbm.at[pl.ds(e0, _K)], esrc_v)
    pltpu.sync_copy(etgt_hbm.at[pl.ds(e0, _K)], etgt_v)
    pltpu.sync_copy(gate_hbm.at[pl.ds(e0, _K)], gate_v)

    def vec(j, c):
      sl = pl.ds(j * 16, 16)
      vals = plsc.load_gather(sup_v, [esrc_v[sl]])
      m_v[sl] = vals * gate_v[sl]
      return c

    lax.fori_loop(0, _K // 16, vec, 0)
    pltpu.sync_copy(m_v, acc_sh.at[etgt_v], add=True)
    return carry

  lax.fori_loop(0, _NCHUNK, chunk, 0)
  plsc.subcore_barrier()

  @pl.when(sid < 10)
  def _writeout():
    pltpu.sync_copy(acc_sh.at[pl.ds(sid * 1000, 1000)], stage_v)
    pltpu.sync_copy(stage_v, out_hbm.at[pl.ds(cid * _N + sid * 1000, 1000)])


def kernel(node_features, edge_features, Esrc, Etgt, batch,
           Wgc_in, bgc_in, Wgc_mid, bgc_mid, Wgc_out, bgc_out,
           We1_in, be1_in, We2_in, be2_in,
           We1_mid, be1_mid, We2_mid, be2_mid,
           We1_out, be1_out, We2_out, be2_out):
  padh = _HP - _H
  f32 = jnp.float32

  def padw(w):
    return jnp.pad(w, ((0, 0), (0, padh)))

  def padw2(w):
    return jnp.pad(w, ((0, padh), (0, padh)))

  def padb(b):
    return jnp.pad(b, (0, padh)).reshape(1, _HP)

  efin, efmid, efout = _edge_mlps(
      edge_features,
      padw(We1_in), padb(be1_in), padw2(We2_in), padb(be2_in),
      padw(We1_mid), padb(be1_mid), padw2(We2_mid), padb(be2_mid),
      We1_out, be1_out.reshape(1, 1), We2_out, be2_out.reshape(1, 1))

  sup_in = _linear(node_features, padw(Wgc_in), padb(bgc_in))

  zer = jnp.zeros((_N, _HP), f32)
  p1 = _sc_layer(sup_in, efin, Esrc, Etgt, zer)
  sup_mid = _relu_sum_linear(p1, padw2(Wgc_mid), padb(bgc_mid))
  p2 = _sc_layer(sup_mid, efmid, Esrc, Etgt, zer)
  sup_out = _relu_sum_linear(
      p2, jnp.pad(Wgc_out, ((0, padh), (0, 0))), bgc_out.reshape(1, 1))

  y3 = _sc_layer_out(sup_out.reshape(_N), efout.reshape(_E), Esrc, Etgt,
                     jnp.zeros((_N,), f32))
  pooled = _pool(batch.reshape(1, _N), y3.reshape(_NC, _N, 1))
  return pooled


# speedup vs baseline: 5.8666x; 1.4368x over previous
import functools

import jax
import jax.numpy as jnp
from jax import lax
from jax.experimental import pallas as pl
from jax.experimental.pallas import tpu as pltpu
from jax.experimental.pallas import tpu_sc as plsc

_N = 10000
_E = 320000
_DF = 128
_DE = 16
_H = 73
_HP = 128
_NG = 64

_NC = 2
_NS = 16
_NW = _NC * _NS
_EPW = _E // _NW
_K = 80
_NCHUNK = _EPW // _K


def _emlp_body(ef_ref, w1i, b1i, w2i, b2i, w1m, b1m, w2m, b2m,
               w1o, b1o, w2o, b2o, efin_ref, efmid_ref, efout_ref):
  ef = ef_ref[...]

  def mlp(w1, b1, w2, b2):
    h = jnp.maximum(
        jnp.dot(ef, w1[...], preferred_element_type=jnp.float32) + b1[...], 0.0)
    z = jnp.dot(h, w2[...], preferred_element_type=jnp.float32) + b2[...]
    return jax.nn.sigmoid(z)

  efin_ref[...] = mlp(w1i, b1i, w2i, b2i)
  efmid_ref[...] = mlp(w1m, b1m, w2m, b2m)
  efout_ref[...] = mlp(w1o, b1o, w2o, b2o)


def _edge_mlps(ef, w1i, b1i, w2i, b2i, w1m, b1m, w2m, b2m, w1o, b1o, w2o, b2o):
  be = 8000
  grid = _E // be
  full = lambda shape: pl.BlockSpec(shape, lambda i: (0, 0))
  return pl.pallas_call(
      _emlp_body,
      grid=(grid,),
      in_specs=[
          pl.BlockSpec((be, _DE), lambda i: (i, 0)),
          full(w1i.shape), full(b1i.shape), full(w2i.shape), full(b2i.shape),
          full(w1m.shape), full(b1m.shape), full(w2m.shape), full(b2m.shape),
          full(w1o.shape), full(b1o.shape), full(w2o.shape), full(b2o.shape),
      ],
      out_specs=[
          pl.BlockSpec((be, _HP), lambda i: (i, 0)),
          pl.BlockSpec((be, _HP), lambda i: (i, 0)),
          pl.BlockSpec((be, 1), lambda i: (i, 0)),
      ],
      out_shape=[
          jax.ShapeDtypeStruct((_E, _HP), jnp.float32),
          jax.ShapeDtypeStruct((_E, _HP), jnp.float32),
          jax.ShapeDtypeStruct((_E, 1), jnp.float32),
      ],
  )(ef, w1i, b1i, w2i, b2i, w1m, b1m, w2m, b2m, w1o, b1o, w2o, b2o)


def _lin_body(x_ref, w_ref, b_ref, o_ref):
  o_ref[...] = (jnp.dot(x_ref[...], w_ref[...],
                        preferred_element_type=jnp.float32) + b_ref[...])


def _linear(x, w, b):
  return pl.pallas_call(
      _lin_body,
      out_shape=jax.ShapeDtypeStruct((x.shape[0], w.shape[1]), jnp.float32),
  )(x, w, b)


def _mid_body(p_ref, w_ref, b_ref, o_ref):
  y = jnp.maximum(p_ref[0] + p_ref[1], 0.0)
  o_ref[...] = (jnp.dot(y, w_ref[...],
                        preferred_element_type=jnp.float32) + b_ref[...])


def _relu_sum_linear(p, w, b):
  return pl.pallas_call(
      _mid_body,
      out_shape=jax.ShapeDtypeStruct((p.shape[1], w.shape[1]), jnp.float32),
  )(p, w, b)


def _pool_body(b_ref, q_ref, o_ref):
  y = q_ref[0] + q_ref[1]
  g = lax.broadcasted_iota(jnp.int32, (_NG, _N), 0)
  m = (g == b_ref[...]).astype(jnp.float32)
  o_ref[...] = jnp.dot(m, y, preferred_element_type=jnp.float32)


def _pool(batch_row, q):
  return pl.pallas_call(
      _pool_body,
      out_shape=jax.ShapeDtypeStruct((_NG, 1), jnp.float32),
  )(batch_row, q)


_MESH = plsc.VectorSubcoreMesh(
    core_axis_name="c", subcore_axis_name="s", num_cores=_NC, num_subcores=_NS)


@functools.partial(
    pl.kernel,
    out_type=jax.ShapeDtypeStruct((_NC, _N, _HP), jnp.float32),
    mesh=_MESH,
    scratch_types=[
        pltpu.VMEM((_EPW,), jnp.int32),
        pltpu.VMEM((_K,), jnp.int32),
        pltpu.VMEM((_K, _HP), jnp.float32),
        pltpu.VMEM((_K, _HP), jnp.float32),
        pltpu.VMEM_SHARED((_N, _HP), jnp.float32),
        pltpu.SemaphoreType.DMA,
    ],
)
def _sc_layer(sup_hbm, gate_hbm, esrc_hbm, etgt_hbm, zer_hbm, out_hbm,
              esrc_all, etgt_v, rows_v, gate_v, acc_sh, sem):
  cid = lax.axis_index("c")
  sid = lax.axis_index("s")
  wid = cid * _NS + sid
  ebase = wid * _EPW

  pltpu.sync_copy(esrc_hbm.at[pl.ds(ebase, _EPW)], esrc_all)

  @pl.when(sid < 10)
  def _zero():
    pltpu.sync_copy(zer_hbm.at[pl.ds(sid * 1000, 1000)],
                    acc_sh.at[pl.ds(sid * 1000, 1000)])

  plsc.subcore_barrier()

  def chunk(i, carry):
    e0 = ebase + i * _K
    h1 = pltpu.async_copy(sup_hbm.at[esrc_all.at[pl.ds(i * _K, _K)]],
                          rows_v, sem)
    h2 = pltpu.async_copy(gate_hbm.at[pl.ds(e0, _K)], gate_v, sem)
    h3 = pltpu.async_copy(etgt_hbm.at[pl.ds(e0, _K)], etgt_v, sem)
    h1.wait()
    h2.wait()
    h3.wait()

    @plsc.parallel_loop(0, _K, unroll=4)
    def edge(j):
      for t in range(5):
        sl = pl.ds(t * 16, 16)
        rows_v[j, sl] = rows_v[j, sl] * gate_v[j, sl]
    pltpu.sync_copy(rows_v, acc_sh.at[etgt_v], add=True)
    return carry

  lax.fori_loop(0, _NCHUNK, chunk, 0)
  plsc.subcore_barrier()

  @pl.when(sid < 10)
  def _writeout():
    pltpu.sync_copy(acc_sh.at[pl.ds(sid * 1000, 1000)],
                    out_hbm.at[cid, pl.ds(sid * 1000, 1000)])


@functools.partial(
    pl.kernel,
    out_type=jax.ShapeDtypeStruct((_NC * _N,), jnp.float32),
    mesh=_MESH,
    compiler_params=pltpu.CompilerParams(needs_layout_passes=False),
    scratch_types=[
        pltpu.VMEM((_N,), jnp.float32),
        pltpu.VMEM((_EPW,), jnp.int32),
        pltpu.VMEM((_EPW,), jnp.float32),
        pltpu.VMEM((_K,), jnp.int32),
        pltpu.VMEM((_K,), jnp.float32),
        pltpu.VMEM((1000,), jnp.float32),
        pltpu.VMEM_SHARED((_N,), jnp.float32),
    ],
)
def _sc_layer_out(sup_hbm, gate_hbm, esrc_hbm, etgt_hbm, zer_hbm, out_hbm,
                  sup_v, esrc_all, gate_all, etgt_v, m_v, stage_v, acc_sh):
  cid = lax.axis_index("c")
  sid = lax.axis_index("s")
  wid = cid * _NS + sid
  ebase = wid * _EPW
  pltpu.sync_copy(sup_hbm, sup_v)
  pltpu.sync_copy(esrc_hbm.at[pl.ds(ebase, _EPW)], esrc_all)
  pltpu.sync_copy(gate_hbm.at[pl.ds(ebase, _EPW)], gate_all)

  @pl.when(sid < 10)
  def _zero():
    pltpu.sync_copy(zer_hbm.at[pl.ds(sid * 1000, 1000)], stage_v)
    pltpu.sync_copy(stage_v, acc_sh.at[pl.ds(sid * 1000, 1000)])

  plsc.subcore_barrier()

  def chunk(i, carry):
    pltpu.sync_copy(etgt_hbm.at[pl.ds(ebase + i * _K, _K)], etgt_v)

    def vec(j, c):
      sl = pl.ds(j * 16, 16)
      idx = esrc_all[pl.ds(i * _K + j * 16, 16)]
      vals = plsc.load_gather(sup_v, [idx])
      m_v[sl] = vals * gate_all[pl.ds(i * _K + j * 16, 16)]
      return c

    lax.fori_loop(0, _K // 16, vec, 0)
    pltpu.sync_copy(m_v, acc_sh.at[etgt_v], add=True)
    return carry

  lax.fori_loop(0, _NCHUNK, chunk, 0)
  plsc.subcore_barrier()

  @pl.when(sid < 10)
  def _writeout():
    pltpu.sync_copy(acc_sh.at[pl.ds(sid * 1000, 1000)], stage_v)
    pltpu.sync_copy(stage_v, out_hbm.at[pl.ds(cid * _N + sid * 1000, 1000)])


def kernel(node_features, edge_features, Esrc, Etgt, batch,
           Wgc_in, bgc_in, Wgc_mid, bgc_mid, Wgc_out, bgc_out,
           We1_in, be1_in, We2_in, be2_in,
           We1_mid, be1_mid, We2_mid, be2_mid,
           We1_out, be1_out, We2_out, be2_out):
  padh = _HP - _H
  f32 = jnp.float32

  def padw(w):
    return jnp.pad(w, ((0, 0), (0, padh)))

  def padw2(w):
    return jnp.pad(w, ((0, padh), (0, padh)))

  def padb(b):
    return jnp.pad(b, (0, padh)).reshape(1, _HP)

  efin, efmid, efout = _edge_mlps(
      edge_features,
      padw(We1_in), padb(be1_in), padw2(We2_in), padb(be2_in),
      padw(We1_mid), padb(be1_mid), padw2(We2_mid), padb(be2_mid),
      We1_out, be1_out.reshape(1, 1), We2_out, be2_out.reshape(1, 1))

  sup_in = _linear(node_features, padw(Wgc_in), padb(bgc_in))

  zer = jnp.zeros((_N, _HP), f32)
  p1 = _sc_layer(sup_in, efin, Esrc, Etgt, zer)
  sup_mid = _relu_sum_linear(p1, padw2(Wgc_mid), padb(bgc_mid))
  p2 = _sc_layer(sup_mid, efmid, Esrc, Etgt, zer)
  sup_out = _relu_sum_linear(
      p2, jnp.pad(Wgc_out, ((0, padh), (0, 0))), bgc_out.reshape(1, 1))

  y3 = _sc_layer_out(sup_out.reshape(_N), efout.reshape(_E), Esrc, Etgt,
                     jnp.zeros((_N,), f32))
  pooled = _pool(batch.reshape(1, _N), y3.reshape(_NC, _N, 1))
  return pooled


# speedup vs baseline: 6.2648x; 1.0679x over previous
import functools

import jax
import jax.numpy as jnp
from jax import lax
from jax.experimental import pallas as pl
from jax.experimental.pallas import tpu as pltpu
from jax.experimental.pallas import tpu_sc as plsc

_N = 10000
_E = 320000
_DF = 128
_DE = 16
_H = 73
_HP = 128
_HG = 80
_NG = 64

_NC = 2
_NS = 16
_NW = _NC * _NS
_EPW = _E // _NW
_K = 80
_NCHUNK = _EPW // _K
_KH = 128
_NFULL = _EPW // _KH
_KTAIL = _EPW - _NFULL * _KH


def _emlp_body(ef_ref, w1i, b1i, w2i, b2i, w1m, b1m, w2m, b2m,
               w1o, b1o, w2o, b2o, efin_ref, efmid_ref, efout_ref):
  ef = ef_ref[...]

  def mlp(w1, b1, w2, b2):
    h = jnp.maximum(
        jnp.dot(ef, w1[...], preferred_element_type=jnp.float32) + b1[...], 0.0)
    z = jnp.dot(h, w2[...], preferred_element_type=jnp.float32) + b2[...]
    return jax.nn.sigmoid(z)

  efin_ref[...] = mlp(w1i, b1i, w2i, b2i)
  efmid_ref[...] = mlp(w1m, b1m, w2m, b2m)
  efout_ref[...] = mlp(w1o, b1o, w2o, b2o)


def _edge_mlps(ef, w1i, b1i, w2i, b2i, w1m, b1m, w2m, b2m, w1o, b1o, w2o, b2o):
  be = 8000
  grid = _E // be
  full = lambda shape: pl.BlockSpec(shape, lambda i: (0, 0))
  return pl.pallas_call(
      _emlp_body,
      grid=(grid,),
      in_specs=[
          pl.BlockSpec((be, _DE), lambda i: (i, 0)),
          full(w1i.shape), full(b1i.shape), full(w2i.shape), full(b2i.shape),
          full(w1m.shape), full(b1m.shape), full(w2m.shape), full(b2m.shape),
          full(w1o.shape), full(b1o.shape), full(w2o.shape), full(b2o.shape),
      ],
      out_specs=[
          pl.BlockSpec((be, _HG), lambda i: (i, 0)),
          pl.BlockSpec((be, _HG), lambda i: (i, 0)),
          pl.BlockSpec((be, 1), lambda i: (i, 0)),
      ],
      out_shape=[
          jax.ShapeDtypeStruct((_E, _HG), jnp.float32),
          jax.ShapeDtypeStruct((_E, _HG), jnp.float32),
          jax.ShapeDtypeStruct((_E, 1), jnp.float32),
      ],
  )(ef, w1i, b1i, w2i, b2i, w1m, b1m, w2m, b2m, w1o, b1o, w2o, b2o)


def _lin_body(x_ref, w_ref, b_ref, o_ref):
  o_ref[...] = (jnp.dot(x_ref[...], w_ref[...],
                        preferred_element_type=jnp.float32) + b_ref[...])


def _linear(x, w, b):
  return pl.pallas_call(
      _lin_body,
      out_shape=jax.ShapeDtypeStruct((x.shape[0], w.shape[1]), jnp.float32),
  )(x, w, b)


def _mid_body(p_ref, w_ref, b_ref, o_ref):
  y = jnp.maximum(p_ref[0] + p_ref[1], 0.0)
  o_ref[...] = (jnp.dot(y, w_ref[...],
                        preferred_element_type=jnp.float32) + b_ref[...])


def _relu_sum_linear(p, w, b):
  return pl.pallas_call(
      _mid_body,
      out_shape=jax.ShapeDtypeStruct((p.shape[1], w.shape[1]), jnp.float32),
  )(p, w, b)


def _pool_body(b_ref, q_ref, o_ref):
  y = q_ref[0] + q_ref[1]
  g = lax.broadcasted_iota(jnp.int32, (_NG, _N), 0)
  m = (g == b_ref[...]).astype(jnp.float32)
  o_ref[...] = jnp.dot(m, y, preferred_element_type=jnp.float32)


def _pool(batch_row, q):
  return pl.pallas_call(
      _pool_body,
      out_shape=jax.ShapeDtypeStruct((_NG, 1), jnp.float32),
  )(batch_row, q)


_MESH = plsc.VectorSubcoreMesh(
    core_axis_name="c", subcore_axis_name="s", num_cores=_NC, num_subcores=_NS)


@functools.partial(
    pl.kernel,
    out_type=jax.ShapeDtypeStruct((_NC, _N, _HP), jnp.float32),
    mesh=_MESH,
    scratch_types=[
        pltpu.VMEM((_EPW,), jnp.int32),
        pltpu.VMEM((_KH,), jnp.int32),
        pltpu.VMEM((_KTAIL,), jnp.int32),
        pltpu.VMEM((_KH, _HP), jnp.float32),
        pltpu.VMEM((_KH, _HG), jnp.float32),
        pltpu.VMEM_SHARED((_N, _HP), jnp.float32),
        pltpu.SemaphoreType.DMA,
    ],
)
def _sc_layer(sup_hbm, gate_hbm, esrc_hbm, etgt_hbm, zer_hbm, out_hbm,
              esrc_all, etgt_v, etgt_t, rows_v, gate_v, acc_sh, sem):
  cid = lax.axis_index("c")
  sid = lax.axis_index("s")
  wid = cid * _NS + sid
  ebase = wid * _EPW

  pltpu.sync_copy(esrc_hbm.at[pl.ds(ebase, _EPW)], esrc_all)

  @pl.when(sid < 10)
  def _zero():
    pltpu.sync_copy(zer_hbm.at[pl.ds(sid * 1000, 1000)],
                    acc_sh.at[pl.ds(sid * 1000, 1000)])

  plsc.subcore_barrier()

  def do_chunk(off, k, etgt_ref):
    h1 = pltpu.async_copy(sup_hbm.at[esrc_all.at[pl.ds(off, k)]],
                          rows_v.at[pl.ds(0, k)], sem)
    h2 = pltpu.async_copy(gate_hbm.at[pl.ds(ebase + off, k)],
                          gate_v.at[pl.ds(0, k)], sem)
    h3 = pltpu.async_copy(etgt_hbm.at[pl.ds(ebase + off, k)], etgt_ref, sem)
    h1.wait()
    h2.wait()
    h3.wait()

    @plsc.parallel_loop(0, k, unroll=4)
    def edge(j):
      for t in range(5):
        sl = pl.ds(t * 16, 16)
        rows_v[j, sl] = rows_v[j, sl] * gate_v[j, sl]

    pltpu.sync_copy(rows_v.at[pl.ds(0, k)], acc_sh.at[etgt_ref], add=True)

  def chunk(i, carry):
    do_chunk(i * _KH, _KH, etgt_v)
    return carry

  lax.fori_loop(0, _NFULL, chunk, 0)
  do_chunk(_NFULL * _KH, _KTAIL, etgt_t)

  plsc.subcore_barrier()

  @pl.when(sid < 10)
  def _writeout():
    pltpu.sync_copy(acc_sh.at[pl.ds(sid * 1000, 1000)],
                    out_hbm.at[cid, pl.ds(sid * 1000, 1000)])


@functools.partial(
    pl.kernel,
    out_type=jax.ShapeDtypeStruct((_NC * _N,), jnp.float32),
    mesh=_MESH,
    compiler_params=pltpu.CompilerParams(needs_layout_passes=False),
    scratch_types=[
        pltpu.VMEM((_N,), jnp.float32),
        pltpu.VMEM((_EPW,), jnp.int32),
        pltpu.VMEM((_EPW,), jnp.float32),
        pltpu.VMEM((_K,), jnp.int32),
        pltpu.VMEM((_K,), jnp.float32),
        pltpu.VMEM((1000,), jnp.float32),
        pltpu.VMEM_SHARED((_N,), jnp.float32),
    ],
)
def _sc_layer_out(sup_hbm, gate_hbm, esrc_hbm, etgt_hbm, zer_hbm, out_hbm,
                  sup_v, esrc_all, gate_all, etgt_v, m_v, stage_v, acc_sh):
  cid = lax.axis_index("c")
  sid = lax.axis_index("s")
  wid = cid * _NS + sid
  ebase = wid * _EPW
  pltpu.sync_copy(sup_hbm, sup_v)
  pltpu.sync_copy(esrc_hbm.at[pl.ds(ebase, _EPW)], esrc_all)
  pltpu.sync_copy(gate_hbm.at[pl.ds(ebase, _EPW)], gate_all)

  @pl.when(sid < 10)
  def _zero():
    pltpu.sync_copy(zer_hbm.at[pl.ds(sid * 1000, 1000)], stage_v)
    pltpu.sync_copy(stage_v, acc_sh.at[pl.ds(sid * 1000, 1000)])

  plsc.subcore_barrier()

  def chunk(i, carry):
    pltpu.sync_copy(etgt_hbm.at[pl.ds(ebase + i * _K, _K)], etgt_v)

    def vec(j, c):
      sl = pl.ds(j * 16, 16)
      idx = esrc_all[pl.ds(i * _K + j * 16, 16)]
      vals = plsc.load_gather(sup_v, [idx])
      m_v[sl] = vals * gate_all[pl.ds(i * _K + j * 16, 16)]
      return c

    lax.fori_loop(0, _K // 16, vec, 0)
    pltpu.sync_copy(m_v, acc_sh.at[etgt_v], add=True)
    return carry

  lax.fori_loop(0, _NCHUNK, chunk, 0)
  plsc.subcore_barrier()

  @pl.when(sid < 10)
  def _writeout():
    pltpu.sync_copy(acc_sh.at[pl.ds(sid * 1000, 1000)], stage_v)
    pltpu.sync_copy(stage_v, out_hbm.at[pl.ds(cid * _N + sid * 1000, 1000)])


def kernel(node_features, edge_features, Esrc, Etgt, batch,
           Wgc_in, bgc_in, Wgc_mid, bgc_mid, Wgc_out, bgc_out,
           We1_in, be1_in, We2_in, be2_in,
           We1_mid, be1_mid, We2_mid, be2_mid,
           We1_out, be1_out, We2_out, be2_out):
  f32 = jnp.float32

  def padw(w, cols):
    return jnp.pad(w, ((0, 0), (0, cols - w.shape[1])))

  def padw2(w, rows, cols):
    return jnp.pad(w, ((0, rows - w.shape[0]), (0, cols - w.shape[1])))

  def padb(b, cols):
    return jnp.pad(b, (0, cols - b.shape[0])).reshape(1, cols)

  efin, efmid, efout = _edge_mlps(
      edge_features,
      padw(We1_in, _HG), padb(be1_in, _HG),
      padw2(We2_in, _HG, _HG), padb(be2_in, _HG),
      padw(We1_mid, _HG), padb(be1_mid, _HG),
      padw2(We2_mid, _HG, _HG), padb(be2_mid, _HG),
      We1_out, be1_out.reshape(1, 1), We2_out, be2_out.reshape(1, 1))

  sup_in = _linear(node_features, padw(Wgc_in, _HP), padb(bgc_in, _HP))

  zer = jnp.zeros((_N, _HP), f32)
  p1 = _sc_layer(sup_in, efin, Esrc, Etgt, zer)
  sup_mid = _relu_sum_linear(p1, padw2(Wgc_mid, _HP, _HP), padb(bgc_mid, _HP))
  p2 = _sc_layer(sup_mid, efmid, Esrc, Etgt, zer)
  sup_out = _relu_sum_linear(
      p2, jnp.pad(Wgc_out, ((0, _HP - _H), (0, 0))), bgc_out.reshape(1, 1))

  y3 = _sc_layer_out(sup_out.reshape(_N), efout.reshape(_E), Esrc, Etgt,
                     jnp.zeros((_N,), f32))
  pooled = _pool(batch.reshape(1, _N), y3.reshape(_NC, _N, 1))
  return pooled


# speedup vs baseline: 6.3554x; 1.0145x over previous
import functools

import jax
import jax.numpy as jnp
from jax import lax
from jax.experimental import pallas as pl
from jax.experimental.pallas import tpu as pltpu
from jax.experimental.pallas import tpu_sc as plsc

_N = 10000
_E = 320000
_DF = 128
_DE = 16
_H = 73
_HP = 128
_HG = 80
_NG = 64

_NC = 2
_NS = 16
_NW = _NC * _NS
_EPW = _E // _NW
_K = 80
_NCHUNK = _EPW // _K
_KH = 128
_NFULL = _EPW // _KH
_KTAIL = _EPW - _NFULL * _KH


def _emlp_body(ef_ref, w1, b1, w2, b2, out_ref):
  ef = ef_ref[...]
  h = jnp.maximum(
      jnp.dot(ef, w1[...], preferred_element_type=jnp.float32) + b1[...], 0.0)
  z = jnp.dot(h, w2[...], preferred_element_type=jnp.float32) + b2[...]
  out_ref[...] = jax.nn.sigmoid(z)


def _edge_mlp(ef, w1, b1, w2, b2):
  be = 8000
  wout = w2.shape[1]
  full = lambda shape: pl.BlockSpec(shape, lambda i: (0, 0))
  return pl.pallas_call(
      _emlp_body,
      grid=(_E // be,),
      in_specs=[
          pl.BlockSpec((be, _DE), lambda i: (i, 0)),
          full(w1.shape), full(b1.shape), full(w2.shape), full(b2.shape),
      ],
      out_specs=pl.BlockSpec((be, wout), lambda i: (i, 0)),
      out_shape=jax.ShapeDtypeStruct((_E, wout), jnp.float32),
  )(ef, w1, b1, w2, b2)


def _lin_body(x_ref, w_ref, b_ref, o_ref):
  o_ref[...] = (jnp.dot(x_ref[...], w_ref[...],
                        preferred_element_type=jnp.float32) + b_ref[...])


def _linear(x, w, b):
  return pl.pallas_call(
      _lin_body,
      out_shape=jax.ShapeDtypeStruct((x.shape[0], w.shape[1]), jnp.float32),
  )(x, w, b)


def _mid_body(p_ref, w_ref, b_ref, o_ref):
  y = jnp.maximum(p_ref[0] + p_ref[1], 0.0)
  o_ref[...] = (jnp.dot(y, w_ref[...],
                        preferred_element_type=jnp.float32) + b_ref[...])


def _relu_sum_linear(p, w, b):
  return pl.pallas_call(
      _mid_body,
      out_shape=jax.ShapeDtypeStruct((p.shape[1], w.shape[1]), jnp.float32),
  )(p, w, b)


def _pool_body(b_ref, q_ref, o_ref):
  y = q_ref[0] + q_ref[1]
  g = lax.broadcasted_iota(jnp.int32, (_NG, _N), 0)
  m = (g == b_ref[...]).astype(jnp.float32)
  o_ref[...] = jnp.dot(m, y, preferred_element_type=jnp.float32)


def _pool(batch_row, q):
  return pl.pallas_call(
      _pool_body,
      out_shape=jax.ShapeDtypeStruct((_NG, 1), jnp.float32),
  )(batch_row, q)


_MESH = plsc.VectorSubcoreMesh(
    core_axis_name="c", subcore_axis_name="s", num_cores=_NC, num_subcores=_NS)


@functools.partial(
    pl.kernel,
    out_type=jax.ShapeDtypeStruct((_NC, _N, _HP), jnp.float32),
    mesh=_MESH,
    scratch_types=[
        pltpu.VMEM((_EPW,), jnp.int32),
        pltpu.VMEM((_KH,), jnp.int32),
        pltpu.VMEM((_KTAIL,), jnp.int32),
        pltpu.VMEM((_KH, _HP), jnp.float32),
        pltpu.VMEM((_KH, _HG), jnp.float32),
        pltpu.VMEM_SHARED((_N, _HP), jnp.float32),
        pltpu.SemaphoreType.DMA,
    ],
)
def _sc_layer(sup_hbm, gate_hbm, esrc_hbm, etgt_hbm, zer_hbm, out_hbm,
              esrc_all, etgt_v, etgt_t, rows_v, gate_v, acc_sh, sem):
  cid = lax.axis_index("c")
  sid = lax.axis_index("s")
  wid = cid * _NS + sid
  ebase = wid * _EPW

  pltpu.sync_copy(esrc_hbm.at[pl.ds(ebase, _EPW)], esrc_all)

  @pl.when(sid < 10)
  def _zero():
    pltpu.sync_copy(zer_hbm.at[pl.ds(sid * 1000, 1000)],
                    acc_sh.at[pl.ds(sid * 1000, 1000)])

  plsc.subcore_barrier()

  def do_chunk(off, k, etgt_ref):
    h1 = pltpu.async_copy(sup_hbm.at[esrc_all.at[pl.ds(off, k)]],
                          rows_v.at[pl.ds(0, k)], sem)
    h2 = pltpu.async_copy(gate_hbm.at[pl.ds(ebase + off, k)],
                          gate_v.at[pl.ds(0, k)], sem)
    h3 = pltpu.async_copy(etgt_hbm.at[pl.ds(ebase + off, k)], etgt_ref, sem)
    h1.wait()
    h2.wait()
    h3.wait()

    @plsc.parallel_loop(0, k, unroll=4)
    def edge(j):
      for t in range(5):
        sl = pl.ds(t * 16, 16)
        rows_v[j, sl] = rows_v[j, sl] * gate_v[j, sl]

    pltpu.sync_copy(rows_v.at[pl.ds(0, k)], acc_sh.at[etgt_ref], add=True)

  def chunk(i, carry):
    do_chunk(i * _KH, _KH, etgt_v)
    return carry

  lax.fori_loop(0, _NFULL, chunk, 0)
  do_chunk(_NFULL * _KH, _KTAIL, etgt_t)

  plsc.subcore_barrier()

  @pl.when(sid < 10)
  def _writeout():
    pltpu.sync_copy(acc_sh.at[pl.ds(sid * 1000, 1000)],
                    out_hbm.at[cid, pl.ds(sid * 1000, 1000)])


@functools.partial(
    pl.kernel,
    out_type=jax.ShapeDtypeStruct((_NC * _N,), jnp.float32),
    mesh=_MESH,
    compiler_params=pltpu.CompilerParams(needs_layout_passes=False),
    scratch_types=[
        pltpu.VMEM((_N,), jnp.float32),
        pltpu.VMEM((_EPW,), jnp.int32),
        pltpu.VMEM((_EPW,), jnp.float32),
        pltpu.VMEM((_K,), jnp.int32),
        pltpu.VMEM((_K,), jnp.float32),
        pltpu.VMEM((1000,), jnp.float32),
        pltpu.VMEM_SHARED((_N,), jnp.float32),
    ],
)
def _sc_layer_out(sup_hbm, gate_hbm, esrc_hbm, etgt_hbm, zer_hbm, out_hbm,
                  sup_v, esrc_all, gate_all, etgt_v, m_v, stage_v, acc_sh):
  cid = lax.axis_index("c")
  sid = lax.axis_index("s")
  wid = cid * _NS + sid
  ebase = wid * _EPW
  pltpu.sync_copy(sup_hbm, sup_v)
  pltpu.sync_copy(esrc_hbm.at[pl.ds(ebase, _EPW)], esrc_all)
  pltpu.sync_copy(gate_hbm.at[pl.ds(ebase, _EPW)], gate_all)

  @pl.when(sid < 10)
  def _zero():
    pltpu.sync_copy(zer_hbm.at[pl.ds(sid * 1000, 1000)], stage_v)
    pltpu.sync_copy(stage_v, acc_sh.at[pl.ds(sid * 1000, 1000)])

  plsc.subcore_barrier()

  def chunk(i, carry):
    pltpu.sync_copy(etgt_hbm.at[pl.ds(ebase + i * _K, _K)], etgt_v)

    def vec(j, c):
      sl = pl.ds(j * 16, 16)
      idx = esrc_all[pl.ds(i * _K + j * 16, 16)]
      vals = plsc.load_gather(sup_v, [idx])
      m_v[sl] = vals * gate_all[pl.ds(i * _K + j * 16, 16)]
      return c

    lax.fori_loop(0, _K // 16, vec, 0)
    pltpu.sync_copy(m_v, acc_sh.at[etgt_v], add=True)
    return carry

  lax.fori_loop(0, _NCHUNK, chunk, 0)
  plsc.subcore_barrier()

  @pl.when(sid < 10)
  def _writeout():
    pltpu.sync_copy(acc_sh.at[pl.ds(sid * 1000, 1000)], stage_v)
    pltpu.sync_copy(stage_v, out_hbm.at[pl.ds(cid * _N + sid * 1000, 1000)])


def kernel(node_features, edge_features, Esrc, Etgt, batch,
           Wgc_in, bgc_in, Wgc_mid, bgc_mid, Wgc_out, bgc_out,
           We1_in, be1_in, We2_in, be2_in,
           We1_mid, be1_mid, We2_mid, be2_mid,
           We1_out, be1_out, We2_out, be2_out):
  f32 = jnp.float32

  def padw(w, cols):
    return jnp.pad(w, ((0, 0), (0, cols - w.shape[1])))

  def padw2(w, rows, cols):
    return jnp.pad(w, ((0, rows - w.shape[0]), (0, cols - w.shape[1])))

  def padb(b, cols):
    return jnp.pad(b, (0, cols - b.shape[0])).reshape(1, cols)

  efin = _edge_mlp(edge_features,
                   padw(We1_in, _HG), padb(be1_in, _HG),
                   padw2(We2_in, _HG, _HG), padb(be2_in, _HG))
  sup_in = _linear(node_features, padw(Wgc_in, _HP), padb(bgc_in, _HP))

  zer = jnp.zeros((_N, _HP), f32)
  p1 = _sc_layer(sup_in, efin, Esrc, Etgt, zer)
  efmid = _edge_mlp(edge_features,
                    padw(We1_mid, _HG), padb(be1_mid, _HG),
                    padw2(We2_mid, _HG, _HG), padb(be2_mid, _HG))
  efout = _edge_mlp(edge_features,
                    We1_out, be1_out.reshape(1, 1),
                    We2_out, be2_out.reshape(1, 1))
  sup_mid = _relu_sum_linear(p1, padw2(Wgc_mid, _HP, _HP), padb(bgc_mid, _HP))
  p2 = _sc_layer(sup_mid, efmid, Esrc, Etgt, zer)
  sup_out = _relu_sum_linear(
      p2, jnp.pad(Wgc_out, ((0, _HP - _H), (0, 0))), bgc_out.reshape(1, 1))

  y3 = _sc_layer_out(sup_out.reshape(_N), efout.reshape(_E), Esrc, Etgt,
                     jnp.zeros((_N,), f32))
  pooled = _pool(batch.reshape(1, _N), y3.reshape(_NC, _N, 1))
  return pooled
